# Initial kernel scaffold; baseline (speedup 1.0000x reference)
import jax
import jax.numpy as jnp
from jax.experimental import pallas as pl


def kernel(idx, emb, targets):
    raise NotImplementedError("write your pallas kernel here")


# speedup vs baseline: 1.1554x; 1.1554x over previous
import functools

import jax
import jax.numpy as jnp
from jax.experimental import pallas as pl
from jax.experimental.pallas import tpu as pltpu

LANE = 128
SUB = 8
NEG = -1e30


def _ceil_to(x, m):
    return (x + m - 1) // m * m


def _fused_kernel(idx_ref, tgt_ref, embh_ref, embt_ref, logits_ref, part_ref,
                  *, n_rows, masked):
    tm = idx_ref.shape[0]
    vp = embh_ref.shape[0]

    idx = idx_ref[...]
    tgt = tgt_ref[...]
    lane = jax.lax.broadcasted_iota(jnp.int32, (tm, vp), 1)
    idx_mask = lane == idx
    onehot = idx_mask.astype(jnp.bfloat16)
    logits = jnp.dot(onehot, embh_ref[...],
                     preferred_element_type=jnp.float32)
    logits_ref[...] = logits

    embt = embt_ref[...]
    m = jnp.max(embt, axis=0, keepdims=True)
    lse_vec = jnp.log(jnp.sum(jnp.exp(embt - m), axis=0, keepdims=True)) + m

    lse_sel = jnp.where(idx_mask, lse_vec, 0.0)
    tgt_sel = jnp.where(lane == tgt, logits, 0.0)
    terms = lse_sel - tgt_sel
    if masked:
        i = pl.program_id(0)
        row = jax.lax.broadcasted_iota(jnp.int32, (tm, 1), 0)
        terms = jnp.where(i * tm + row < n_rows, terms, 0.0)
    part_ref[...] = jnp.zeros(part_ref.shape, jnp.float32) + jnp.sum(terms)


def kernel(idx, emb, targets):
    B, T = idx.shape
    V = emb.shape[0]
    N = B * T

    vp = _ceil_to(V, LANE)
    tm = min(2048, _ceil_to(N, SUB))
    if _ceil_to(N, tm) // tm < 2 and N > SUB:
        tm = _ceil_to((N + 1) // 2, SUB)
    n_pad = _ceil_to(N, tm)
    num_tiles = n_pad // tm

    emb32 = emb.astype(jnp.float32)
    if vp == V:
        emb_hi = emb32.astype(jnp.bfloat16)
        emb_t = emb32.T
    else:
        emb_hi = jnp.zeros((vp, vp), jnp.bfloat16).at[:V, :V].set(
            emb32.astype(jnp.bfloat16))
        emb_t = jnp.full((vp, vp), NEG, jnp.float32).at[:V, :V].set(emb32.T)

    idx_flat = idx.reshape(-1).astype(jnp.int32)
    tgt_flat = targets.reshape(-1).astype(jnp.int32)
    if n_pad != N:
        idx_flat = jnp.pad(idx_flat, (0, n_pad - N))
        tgt_flat = jnp.pad(tgt_flat, (0, n_pad - N))
    idx_p = idx_flat.reshape(n_pad, 1)
    tgt_p = tgt_flat.reshape(n_pad, 1)

    row_spec = pl.BlockSpec((tm, 1), lambda i: (i, 0))
    tbl_spec = pl.BlockSpec((vp, vp), lambda i: (0, 0))
    logit_spec = pl.BlockSpec((tm, vp), lambda i: (i, 0))
    part_spec = pl.BlockSpec((SUB, LANE), lambda i: (i, 0))

    logits_p, parts = pl.pallas_call(
        functools.partial(_fused_kernel, n_rows=N, masked=(n_pad != N)),
        out_shape=(jax.ShapeDtypeStruct((n_pad, vp), jnp.float32),
                   jax.ShapeDtypeStruct((num_tiles * SUB, LANE), jnp.float32)),
        grid=(num_tiles,),
        in_specs=[row_spec, row_spec, tbl_spec, tbl_spec],
        out_specs=(logit_spec, part_spec),
        compiler_params=pltpu.CompilerParams(
            dimension_semantics=("parallel",)),
    )(idx_p, tgt_p, emb_hi, emb_t)

    logits = logits_p if (n_pad == N and vp == V) else logits_p[:N, :V]
    loss = jnp.sum(parts) / jnp.float32(SUB * LANE) / jnp.float32(N)
    return logits, loss


# speedup vs baseline: 8.2835x; 7.1695x over previous
import functools

import jax
import jax.numpy as jnp
from jax.experimental import pallas as pl
from jax.experimental.pallas import tpu as pltpu

LANE = 128
SUB = 8
NEG = -1e30
CH = 1024


def _ceil_to(x, m):
    return (x + m - 1) // m * m


def _fused_kernel(idx_ref, tgt_ref, embh_ref, embf_ref, logits_ref, part_ref,
                  *, n_rows, masked):
    vp = embh_ref.shape[0]
    ch = idx_ref.shape[2]
    chunks = idx_ref.shape[1]
    tm = chunks * ch

    embh = embh_ref[...]
    embf = embf_ref[...]
    m = jnp.max(embf, axis=1, keepdims=True)
    lse_col = jnp.log(jnp.sum(jnp.exp(embf - m), axis=1,
                              keepdims=True)) + m

    viota = jax.lax.broadcasted_iota(jnp.int32, (vp, ch), 0)
    idx3 = idx_ref[0]
    tgt3 = tgt_ref[0]
    c_acc = jnp.zeros((vp, vp), jnp.float32)
    lse_acc = jnp.float32(0.0)
    for s in range(chunks):
        idx_s = idx3[s:s + 1, :]
        tgt_s = tgt3[s:s + 1, :]
        mi = viota == idx_s
        if masked:
            i = pl.program_id(0)
            liota = jax.lax.broadcasted_iota(jnp.int32, (1, ch), 1)
            mi = jnp.logical_and(mi, (i * tm + s * ch + liota) < n_rows)
        ohi = mi.astype(jnp.bfloat16)
        oht = (viota == tgt_s).astype(jnp.bfloat16)
        logits_s = jax.lax.dot_general(
            ohi, embh, (((0,), (0,)), ((), ())),
            preferred_element_type=jnp.float32)
        logits_ref[s * ch:(s + 1) * ch, :] = logits_s
        c_acc = c_acc + jax.lax.dot_general(
            ohi, oht, (((1,), (1,)), ((), ())),
            preferred_element_type=jnp.float32)
        lse_acc = lse_acc + jnp.sum(jnp.where(mi, lse_col, 0.0))
    part = lse_acc - jnp.sum(c_acc * embf)
    part_ref[...] = jnp.zeros(part_ref.shape, jnp.float32) + part


def kernel(idx, emb, targets):
    B, T = idx.shape
    V = emb.shape[0]
    N = B * T

    vp = _ceil_to(V, LANE)
    tm = SUB * CH
    n_pad = _ceil_to(N, tm)
    num_tiles = n_pad // tm

    emb32 = emb.astype(jnp.float32)
    if vp == V:
        embh = emb32.astype(jnp.bfloat16)
        embf = emb32
    else:
        embh = jnp.zeros((vp, vp), jnp.bfloat16).at[:V, :V].set(
            emb32.astype(jnp.bfloat16))
        embf = jnp.full((vp, vp), NEG, jnp.float32).at[:V, :V].set(emb32)

    idx_flat = idx.reshape(-1).astype(jnp.int32)
    tgt_flat = targets.reshape(-1).astype(jnp.int32)
    if n_pad != N:
        idx_flat = jnp.pad(idx_flat, (0, n_pad - N))
        tgt_flat = jnp.pad(tgt_flat, (0, n_pad - N))
    idx_p = idx_flat.reshape(num_tiles, SUB, CH)
    tgt_p = tgt_flat.reshape(num_tiles, SUB, CH)

    row_spec = pl.BlockSpec((1, SUB, CH), lambda i: (i, 0, 0))
    tbl_spec = pl.BlockSpec((vp, vp), lambda i: (0, 0))
    logit_spec = pl.BlockSpec((tm, vp), lambda i: (i, 0))
    part_spec = pl.BlockSpec((SUB, LANE), lambda i: (i, 0))

    logits_p, parts = pl.pallas_call(
        functools.partial(_fused_kernel, n_rows=N, masked=(n_pad != N)),
        out_shape=(jax.ShapeDtypeStruct((n_pad, vp), jnp.float32),
                   jax.ShapeDtypeStruct((num_tiles * SUB, LANE), jnp.float32)),
        grid=(num_tiles,),
        in_specs=[row_spec, row_spec, tbl_spec, tbl_spec],
        out_specs=(logit_spec, part_spec),
        compiler_params=pltpu.CompilerParams(
            dimension_semantics=("parallel",)),
    )(idx_p, tgt_p, embh, embf)

    logits = logits_p if (n_pad == N and vp == V) else logits_p[:N, :V]
    loss = jnp.sum(parts) / jnp.float32(SUB * LANE) / jnp.float32(N)
    return logits, loss


# speedup vs baseline: 9.4407x; 1.1397x over previous
import functools

import jax
import jax.numpy as jnp
from jax.experimental import pallas as pl
from jax.experimental.pallas import tpu as pltpu

LANE = 128
SUB = 8
NEG = -1e30
CH = 1024


def _ceil_to(x, m):
    return (x + m - 1) // m * m


def _fused_kernel(idx_ref, tgt_ref, embh_ref, embf_ref, logits_ref, part_ref,
                  *, n_rows, masked):
    vp = embh_ref.shape[0]
    ch = idx_ref.shape[2]
    chunks = idx_ref.shape[1]
    tm = chunks * ch

    embh = embh_ref[...]
    embf = embf_ref[...]
    m = jnp.max(embf, axis=1, keepdims=True)
    lse_col = jnp.log(jnp.sum(jnp.exp(embf - m), axis=1,
                              keepdims=True)) + m

    viota = jax.lax.broadcasted_iota(jnp.int32, (vp, ch), 0)
    idx3 = idx_ref[0]
    tgt3 = tgt_ref[0]
    c_acc = jnp.zeros((vp, vp), jnp.float32)
    for s in range(chunks):
        idx_s = idx3[s:s + 1, :]
        tgt_s = tgt3[s:s + 1, :]
        mi = viota == idx_s
        if masked:
            i = pl.program_id(0)
            liota = jax.lax.broadcasted_iota(jnp.int32, (1, ch), 1)
            mi = jnp.logical_and(mi, (i * tm + s * ch + liota) < n_rows)
        ohi = mi.astype(jnp.bfloat16)
        oht = (viota == tgt_s).astype(jnp.bfloat16)
        logits_s = jax.lax.dot_general(
            ohi, embh, (((0,), (0,)), ((), ())),
            preferred_element_type=jnp.float32)
        logits_ref[s * ch:(s + 1) * ch, :] = logits_s
        c_acc = c_acc + jax.lax.dot_general(
            ohi, oht, (((1,), (1,)), ((), ())),
            preferred_element_type=jnp.float32)
    part = jnp.sum(c_acc * (lse_col - embf))
    part_ref[...] = jnp.zeros(part_ref.shape, jnp.float32) + part


def kernel(idx, emb, targets):
    B, T = idx.shape
    V = emb.shape[0]
    N = B * T

    vp = _ceil_to(V, LANE)
    tm = SUB * CH
    n_pad = _ceil_to(N, tm)
    num_tiles = n_pad // tm

    emb32 = emb.astype(jnp.float32)
    if vp == V:
        embh = emb32.astype(jnp.bfloat16)
        embf = emb32
    else:
        embh = jnp.zeros((vp, vp), jnp.bfloat16).at[:V, :V].set(
            emb32.astype(jnp.bfloat16))
        embf = jnp.full((vp, vp), NEG, jnp.float32).at[:V, :V].set(emb32)

    idx_flat = idx.reshape(-1).astype(jnp.int32)
    tgt_flat = targets.reshape(-1).astype(jnp.int32)
    if n_pad != N:
        idx_flat = jnp.pad(idx_flat, (0, n_pad - N))
        tgt_flat = jnp.pad(tgt_flat, (0, n_pad - N))
    idx_p = idx_flat.reshape(num_tiles, SUB, CH)
    tgt_p = tgt_flat.reshape(num_tiles, SUB, CH)

    row_spec = pl.BlockSpec((1, SUB, CH), lambda i: (i, 0, 0))
    tbl_spec = pl.BlockSpec((vp, vp), lambda i: (0, 0))
    logit_spec = pl.BlockSpec((tm, vp), lambda i: (i, 0))
    part_spec = pl.BlockSpec((SUB, LANE), lambda i: (i, 0))

    logits_p, parts = pl.pallas_call(
        functools.partial(_fused_kernel, n_rows=N, masked=(n_pad != N)),
        out_shape=(jax.ShapeDtypeStruct((n_pad, vp), jnp.float32),
                   jax.ShapeDtypeStruct((num_tiles * SUB, LANE), jnp.float32)),
        grid=(num_tiles,),
        in_specs=[row_spec, row_spec, tbl_spec, tbl_spec],
        out_specs=(logit_spec, part_spec),
        compiler_params=pltpu.CompilerParams(
            dimension_semantics=("parallel",)),
    )(idx_p, tgt_p, embh, embf)

    logits = logits_p if (n_pad == N and vp == V) else logits_p[:N, :V]
    loss = jnp.sum(parts) / jnp.float32(SUB * LANE) / jnp.float32(N)
    return logits, loss


# speedup vs baseline: 9.8110x; 1.0392x over previous
import functools

import jax
import jax.numpy as jnp
from jax.experimental import pallas as pl
from jax.experimental.pallas import tpu as pltpu

LANE = 128
SUB = 8
NEG = -1e30
CH = 2048


def _ceil_to(x, m):
    return (x + m - 1) // m * m


def _fused_kernel(idx_ref, tgt_ref, embh_ref, embf_ref, logits_ref, part_ref,
                  *, n_rows, masked):
    vp = embh_ref.shape[0]
    ch = idx_ref.shape[2]
    chunks = idx_ref.shape[1]
    tm = chunks * ch

    embh = embh_ref[...]
    embf = embf_ref[...]
    m = jnp.max(embf, axis=1, keepdims=True)
    lse_col = jnp.log(jnp.sum(jnp.exp(embf - m), axis=1,
                              keepdims=True)) + m

    viota = jax.lax.broadcasted_iota(jnp.int32, (vp, ch), 0)
    idx3 = idx_ref[0]
    tgt3 = tgt_ref[0]
    c_acc = jnp.zeros((vp, vp), jnp.float32)
    for s in range(chunks):
        idx_s = idx3[s:s + 1, :]
        tgt_s = tgt3[s:s + 1, :]
        mi = viota == idx_s
        if masked:
            i = pl.program_id(0)
            liota = jax.lax.broadcasted_iota(jnp.int32, (1, ch), 1)
            mi = jnp.logical_and(mi, (i * tm + s * ch + liota) < n_rows)
        ohi = mi.astype(jnp.bfloat16)
        oht = (viota == tgt_s).astype(jnp.bfloat16)
        logits_s = jax.lax.dot_general(
            ohi, embh, (((0,), (0,)), ((), ())),
            preferred_element_type=jnp.float32)
        logits_ref[s * ch:(s + 1) * ch, :] = logits_s
        c_acc = c_acc + jax.lax.dot_general(
            ohi, oht, (((1,), (1,)), ((), ())),
            preferred_element_type=jnp.float32)
    part = jnp.sum(c_acc * (lse_col - embf))
    part_ref[...] = jnp.zeros(part_ref.shape, jnp.float32) + part


def kernel(idx, emb, targets):
    B, T = idx.shape
    V = emb.shape[0]
    N = B * T

    vp = _ceil_to(V, LANE)
    tm = SUB * CH
    n_pad = _ceil_to(N, tm)
    num_tiles = n_pad // tm

    emb32 = emb.astype(jnp.float32)
    if vp == V:
        embh = emb32.astype(jnp.bfloat16)
        embf = emb32
    else:
        embh = jnp.zeros((vp, vp), jnp.bfloat16).at[:V, :V].set(
            emb32.astype(jnp.bfloat16))
        embf = jnp.full((vp, vp), NEG, jnp.float32).at[:V, :V].set(emb32)

    idx_flat = idx.reshape(-1).astype(jnp.int32)
    tgt_flat = targets.reshape(-1).astype(jnp.int32)
    if n_pad != N:
        idx_flat = jnp.pad(idx_flat, (0, n_pad - N))
        tgt_flat = jnp.pad(tgt_flat, (0, n_pad - N))
    idx_p = idx_flat.reshape(num_tiles, SUB, CH)
    tgt_p = tgt_flat.reshape(num_tiles, SUB, CH)

    row_spec = pl.BlockSpec((1, SUB, CH), lambda i: (i, 0, 0))
    tbl_spec = pl.BlockSpec((vp, vp), lambda i: (0, 0))
    logit_spec = pl.BlockSpec((tm, vp), lambda i: (i, 0))
    part_spec = pl.BlockSpec((SUB, LANE), lambda i: (i, 0))

    logits_p, parts = pl.pallas_call(
        functools.partial(_fused_kernel, n_rows=N, masked=(n_pad != N)),
        out_shape=(jax.ShapeDtypeStruct((n_pad, vp), jnp.float32),
                   jax.ShapeDtypeStruct((num_tiles * SUB, LANE), jnp.float32)),
        grid=(num_tiles,),
        in_specs=[row_spec, row_spec, tbl_spec, tbl_spec],
        out_specs=(logit_spec, part_spec),
        compiler_params=pltpu.CompilerParams(
            dimension_semantics=("parallel",)),
    )(idx_p, tgt_p, embh, embf)

    logits = logits_p if (n_pad == N and vp == V) else logits_p[:N, :V]
    loss = jnp.sum(parts) / jnp.float32(SUB * LANE) / jnp.float32(N)
    return logits, loss
